# baseline (speedup 1.0000x reference)
import functools

import jax
import jax.numpy as jnp
from jax import lax
from jax.experimental import pallas as pl
from jax.experimental.pallas import tpu as pltpu
from jax.experimental.pallas import tpu_sc as plsc

N_USERS_C = 1000000
N_ITEMS_C = 1000000
FEATS = 16
BATCH_C = 16384

_info = plsc.get_sparse_core_info()
NC = _info.num_cores
NS = _info.num_subcores
LANES = _info.num_lanes
NW = NC * NS
B_PER_W = BATCH_C // NW
CHUNK = 128
N_CHUNKS = B_PER_W // CHUNK
GROUPS = B_PER_W // LANES


def _mf_kernel(user_hbm, item_hbm, ub_hbm, ib_hbm, ue_hbm, ie_hbm, out_hbm,
               uidx_v, iidx_v, ub_v, ib_v, urows_v, irows_v, out_v, sem):
    wid = lax.axis_index("s") * NC + lax.axis_index("c")
    base = wid * B_PER_W

    pltpu.sync_copy(user_hbm.at[pl.ds(base, B_PER_W)], uidx_v)
    pltpu.sync_copy(item_hbm.at[pl.ds(base, B_PER_W)], iidx_v)

    copies = []
    for c in range(N_CHUNKS):
        s = pl.ds(c * CHUNK, CHUNK)
        copies.append(pltpu.make_async_copy(ue_hbm.at[uidx_v.at[s]], urows_v.at[s], sem))
        copies.append(pltpu.make_async_copy(ie_hbm.at[iidx_v.at[s]], irows_v.at[s], sem))
        copies.append(pltpu.make_async_copy(ub_hbm.at[uidx_v.at[s]], ub_v.at[s], sem))
        copies.append(pltpu.make_async_copy(ib_hbm.at[iidx_v.at[s]], ib_v.at[s], sem))
    for cp in copies:
        cp.start()
    for cp in copies:
        cp.wait()

    lane_iota = lax.broadcasted_iota(jnp.int32, (LANES,), 0)

    def group_body(g, _):
        rows = lane_iota + g * LANES
        acc = ub_v[pl.ds(g * LANES, LANES)] + ib_v[pl.ds(g * LANES, LANES)]
        for f in range(FEATS):
            col = jnp.full((LANES,), f, jnp.int32)
            uf = plsc.load_gather(urows_v, [rows, col])
            vf = plsc.load_gather(irows_v, [rows, col])
            acc = acc + uf * vf
        out_v[pl.ds(g * LANES, LANES)] = acc
        return 0

    lax.fori_loop(0, GROUPS, group_body, 0)

    pltpu.sync_copy(out_v, out_hbm.at[pl.ds(base, B_PER_W)])


@jax.jit
def _mf(user, item, u_bias_flat, i_bias_flat, u_embed, i_embed):
    mesh = plsc.VectorSubcoreMesh(core_axis_name="c", subcore_axis_name="s")
    return pl.kernel(
        _mf_kernel,
        out_type=jax.ShapeDtypeStruct((BATCH_C,), jnp.float32),
        mesh=mesh,
        compiler_params=pltpu.CompilerParams(needs_layout_passes=False, use_tc_tiling_on_sc=False),
        scratch_types=[
            pltpu.VMEM((B_PER_W,), jnp.int32),
            pltpu.VMEM((B_PER_W,), jnp.int32),
            pltpu.VMEM((B_PER_W,), jnp.float32),
            pltpu.VMEM((B_PER_W,), jnp.float32),
            pltpu.VMEM((B_PER_W, FEATS), jnp.float32),
            pltpu.VMEM((B_PER_W, FEATS), jnp.float32),
            pltpu.VMEM((B_PER_W,), jnp.float32),
            pltpu.SemaphoreType.DMA,
        ],
    )(user, item, u_bias_flat, i_bias_flat, u_embed, i_embed)


def kernel(user, item, u_bias, i_bias, u_embed, i_embed):
    return _mf(
        user.astype(jnp.int32),
        item.astype(jnp.int32),
        u_bias.reshape(-1),
        i_bias.reshape(-1),
        u_embed,
        i_embed,
    )
